# Initial kernel scaffold; baseline (speedup 1.0000x reference)
import jax
import jax.numpy as jnp
from jax.experimental import pallas as pl


def kernel(h_identity, h_others, codebook_identity, codebook_others, Wq_id, bq_id, Wpq_id, bpq_id, Wq_ot, bq_ot, Wpq_ot, bpq_ot):
    raise NotImplementedError("write your pallas kernel here")


# speedup vs baseline: 1.7089x; 1.7089x over previous
import functools

import jax
import jax.numpy as jnp
from jax import lax
from jax.experimental import pallas as pl


def _proj_codebook_kernel(Wpq_ref, cb_ref, bpq_ref, pt_ref):
    pt = lax.dot_general(Wpq_ref[...], cb_ref[...],
                         (((1,), (1,)), ((), ())),
                         preferred_element_type=jnp.float32)
    pt_ref[...] = pt + bpq_ref[...]


def _vq_branch_kernel(h_ref, Wq_ref, bq_ref, cb_ref, pt_ref,
                      out_ref, idx_ref, dsum_ref, *, K):
    h = h_ref[0]
    z = lax.dot_general(Wq_ref[...], h, (((1,), (0,)), ((), ())),
                        preferred_element_type=jnp.float32)
    z = z + bq_ref[...]
    cb = cb_ref[...]
    cn = jnp.sum(cb * cb, axis=1, keepdims=True)
    zn = jnp.sum(z * z, axis=0, keepdims=True)
    s = lax.dot_general(cb, z, (((1,), (0,)), ((), ())),
                        preferred_element_type=jnp.float32)
    d = (zn + cn) - 2.0 * s
    dmin = jnp.min(d, axis=0, keepdims=True)
    ii = lax.broadcasted_iota(jnp.int32, d.shape, 0)
    idx = jnp.min(jnp.where(d == dmin, ii, K), axis=0, keepdims=True)
    idx_ref[0] = idx
    dsum_ref[...] = jnp.sum(dmin, axis=1, keepdims=True)[None]
    onehot = (ii == idx).astype(jnp.float32)
    out_ref[0] = lax.dot_general(pt_ref[...], onehot, (((1,), (0,)), ((), ())),
                                 preferred_element_type=jnp.float32)


def _run_branch(h, Wq, bq, cb, pt):
    B, C_in, H, W = h.shape
    HW = H * W
    emb = Wq.shape[0]
    C_out = pt.shape[0]
    K = cb.shape[0]
    h3 = h.reshape(B, C_in, HW)
    bq_col = bq.reshape(emb, 1)
    out, idx, dsum = pl.pallas_call(
        functools.partial(_vq_branch_kernel, K=K),
        grid=(B,),
        in_specs=[
            pl.BlockSpec((1, C_in, HW), lambda b: (b, 0, 0)),
            pl.BlockSpec((emb, C_in), lambda b: (0, 0)),
            pl.BlockSpec((emb, 1), lambda b: (0, 0)),
            pl.BlockSpec((K, emb), lambda b: (0, 0)),
            pl.BlockSpec((C_out, K), lambda b: (0, 0)),
        ],
        out_specs=[
            pl.BlockSpec((1, C_out, HW), lambda b: (b, 0, 0)),
            pl.BlockSpec((1, 1, HW), lambda b: (b, 0, 0)),
            pl.BlockSpec((1, 1, 1), lambda b: (b, 0, 0)),
        ],
        out_shape=[
            jax.ShapeDtypeStruct((B, C_out, HW), jnp.float32),
            jax.ShapeDtypeStruct((B, 1, HW), jnp.int32),
            jax.ShapeDtypeStruct((B, 1, 1), jnp.float32),
        ],
    )(h3, Wq, bq_col, cb, pt)
    loss = jnp.sum(dsum) * (1.25 / (B * HW * emb))
    return (out.reshape(B, C_out, H, W), loss,
            idx.reshape(B, H, W))


def _proj_codebook(Wpq, cb, bpq):
    C_out, K = Wpq.shape[0], cb.shape[0]
    return pl.pallas_call(
        _proj_codebook_kernel,
        out_shape=jax.ShapeDtypeStruct((C_out, K), jnp.float32),
    )(Wpq, cb, bpq.reshape(C_out, 1))


def kernel(h_identity, h_others, codebook_identity, codebook_others,
           Wq_id, bq_id, Wpq_id, bpq_id, Wq_ot, bq_ot, Wpq_ot, bpq_ot):
    pt_id = _proj_codebook(Wpq_id, codebook_identity, bpq_id)
    pt_ot = _proj_codebook(Wpq_ot, codebook_others, bpq_ot)
    out_id, loss_id, idx_id = _run_branch(
        h_identity, Wq_id, bq_id, codebook_identity, pt_id)
    out_ot, loss_ot, idx_ot = _run_branch(
        h_others, Wq_ot, bq_ot, codebook_others, pt_ot)
    return out_id, out_ot, loss_id, loss_ot, idx_id, idx_ot


# speedup vs baseline: 1.7708x; 1.0362x over previous
import functools

import jax
import jax.numpy as jnp
from jax import lax
from jax.experimental import pallas as pl
from jax.experimental.pallas import tpu as pltpu
from jax.experimental.pallas import tpu_sc as plsc


def _proj_codebook_kernel(cb_ref, Wpq_ref, bpq_ref, p_ref):
    p = lax.dot_general(cb_ref[...], Wpq_ref[...],
                        (((1,), (1,)), ((), ())),
                        preferred_element_type=jnp.float32)
    p_ref[...] = p + bpq_ref[...]


def _proj_codebook(cb, Wpq, bpq):
    C_out, K = Wpq.shape[0], cb.shape[0]
    return pl.pallas_call(
        _proj_codebook_kernel,
        out_shape=jax.ShapeDtypeStruct((K, C_out), jnp.float32),
    )(cb, Wpq, bpq.reshape(1, C_out))


def _vq_branch_kernel(h_ref, Wq_ref, bq_ref, cb_ref, idx_ref, dsum_ref, *, K):
    h = h_ref[0]
    z = lax.dot_general(Wq_ref[...], h, (((1,), (0,)), ((), ())),
                        preferred_element_type=jnp.float32)
    z = z + bq_ref[...]
    cb = cb_ref[...]
    cn = jnp.sum(cb * cb, axis=1, keepdims=True)
    zn = jnp.sum(z * z, axis=0, keepdims=True)
    s = lax.dot_general(cb, z, (((1,), (0,)), ((), ())),
                        preferred_element_type=jnp.float32)
    d = (zn + cn) - 2.0 * s
    dmin = jnp.min(d, axis=0, keepdims=True)
    ii = lax.broadcasted_iota(jnp.int32, d.shape, 0)
    idx_ref[0] = jnp.min(jnp.where(d == dmin, ii, K), axis=0, keepdims=True)
    dsum_ref[...] = jnp.sum(dmin, axis=1, keepdims=True)[None]


def _vq_argmin(h3, Wq, bq, cb):
    B, C_in, HW = h3.shape
    emb = Wq.shape[0]
    K = cb.shape[0]
    idx, dsum = pl.pallas_call(
        functools.partial(_vq_branch_kernel, K=K),
        grid=(B,),
        in_specs=[
            pl.BlockSpec((1, C_in, HW), lambda b: (b, 0, 0)),
            pl.BlockSpec((emb, C_in), lambda b: (0, 0)),
            pl.BlockSpec((emb, 1), lambda b: (0, 0)),
            pl.BlockSpec((K, emb), lambda b: (0, 0)),
        ],
        out_specs=[
            pl.BlockSpec((1, 1, HW), lambda b: (b, 0, 0)),
            pl.BlockSpec((1, 1, 1), lambda b: (b, 0, 0)),
        ],
        out_shape=[
            jax.ShapeDtypeStruct((B, 1, HW), jnp.int32),
            jax.ShapeDtypeStruct((B, 1, 1), jnp.float32),
        ],
    )(h3, Wq, bq.reshape(emb, 1), cb)
    return idx, dsum


def _sc_gather_rows(table, idx_flat):
    N = idx_flat.shape[0]
    D = table.shape[1]
    info = plsc.get_sparse_core_info()
    NW = info.num_cores * info.num_subcores
    n_per_w = N // NW
    CHUNK = 128
    n_chunks = n_per_w // CHUNK
    idx2 = idx_flat.reshape(N // CHUNK, CHUNK)
    mesh = plsc.VectorSubcoreMesh(core_axis_name="c", subcore_axis_name="s")

    @functools.partial(
        pl.kernel, mesh=mesh,
        out_type=jax.ShapeDtypeStruct((N, D), jnp.float32),
        scratch_types=[
            pltpu.VMEM((n_chunks, CHUNK), jnp.int32),
            pltpu.VMEM((n_per_w, D), jnp.float32),
            pltpu.SemaphoreType.DMA,
        ],
    )
    def k(table_hbm, idx_hbm, out_hbm, idx_v, rows_v, sem):
        wid = lax.axis_index("s") * info.num_cores + lax.axis_index("c")
        pltpu.sync_copy(idx_hbm.at[pl.ds(wid * n_chunks, n_chunks)], idx_v)
        copies = [
            pltpu.async_copy(table_hbm.at[idx_v.at[j]],
                             rows_v.at[pl.ds(j * CHUNK, CHUNK)], sem)
            for j in range(n_chunks)
        ]
        for c in copies:
            c.wait()
        pltpu.sync_copy(rows_v, out_hbm.at[pl.ds(wid * n_per_w, n_per_w)])

    return k(table, idx2)


def _run_branch(h, Wq, bq, cb, Wpq, bpq):
    B, C_in, H, W = h.shape
    HW = H * W
    emb = Wq.shape[0]
    C_out = Wpq.shape[0]
    p = _proj_codebook(cb, Wpq, bpq)
    idx, dsum = _vq_argmin(h.reshape(B, C_in, HW), Wq, bq, cb)
    rows = _sc_gather_rows(p, idx.reshape(B * HW))
    out = rows.reshape(B, HW, C_out).transpose(0, 2, 1).reshape(B, C_out, H, W)
    loss = jnp.sum(dsum) * (1.25 / (B * HW * emb))
    return out, loss, idx.reshape(B, H, W)


def kernel(h_identity, h_others, codebook_identity, codebook_others,
           Wq_id, bq_id, Wpq_id, bpq_id, Wq_ot, bq_ot, Wpq_ot, bpq_ot):
    out_id, loss_id, idx_id = _run_branch(
        h_identity, Wq_id, bq_id, codebook_identity, Wpq_id, bpq_id)
    out_ot, loss_ot, idx_ot = _run_branch(
        h_others, Wq_ot, bq_ot, codebook_others, Wpq_ot, bpq_ot)
    return out_id, out_ot, loss_id, loss_ot, idx_id, idx_ot
